# Initial kernel scaffold; baseline (speedup 1.0000x reference)
import functools

import jax
import jax.numpy as jnp
from jax import lax
from jax.experimental import pallas as pl
from jax.experimental.pallas import tpu as pltpu
from jax.experimental.pallas import tpu_sc as plsc

NC = 2
NS = 16
NW = NC * NS
LANES = 16
GROUP = 128
CHUNK = 512
GPC = CHUNK // GROUP


def _build(B, L, D):
    flat = B * L
    per_w = flat // NW
    nchunk = per_w // CHUNK
    rep = L * -(-(CHUNK + L) // L)
    half = D // LANES

    mesh = plsc.VectorSubcoreMesh(core_axis_name="c", subcore_axis_name="s")

    @functools.partial(
        pl.kernel,
        out_type=jax.ShapeDtypeStruct((flat, D), jnp.float32),
        mesh=mesh,
        scratch_types=[
            pltpu.VMEM((GPC, GROUP), jnp.int32),
            pltpu.VMEM((GPC, GROUP), jnp.int32),
            pltpu.VMEM((CHUNK, D), jnp.float32),
            pltpu.VMEM((CHUNK, D), jnp.float32),
            pltpu.VMEM((rep, D), jnp.float32),
            pltpu.SemaphoreType.DMA,
            pltpu.SemaphoreType.DMA,
            pltpu.SemaphoreType.DMA,
            pltpu.SemaphoreType.DMA,
        ],
    )
    def emb(seq_hbm, tok_hbm, pos_hbm, out_hbm,
            idx_a, idx_b, rows_a, rows_b, posrep,
            gsem_a, gsem_b, wsem_a, wsem_b):
        wid = lax.axis_index("s") * NC + lax.axis_index("c")
        row0 = wid * (per_w // GROUP)
        base = wid * per_w

        for j in range(rep // L):
            pltpu.sync_copy(pos_hbm, posrep.at[pl.ds(j * L, L)])

        def fire(c, idx_v, rows_v, gsem):
            pltpu.sync_copy(seq_hbm.at[pl.ds(row0 + c * GPC, GPC)], idx_v)
            return [
                pltpu.async_copy(tok_hbm.at[idx_v.at[g]],
                                 rows_v.at[pl.ds(g * GROUP, GROUP)], gsem)
                for g in range(GPC)
            ]

        def add_store(c, rows_v, wsem):
            p = lax.rem(c * CHUNK, L)

            def ab(i, carry):
                for h in range(half):
                    sl = pl.ds(h * LANES, LANES)
                    rows_v[i, sl] = rows_v[i, sl] + posrep[p + i, sl]
                return carry

            lax.fori_loop(0, CHUNK, ab, 0)
            return pltpu.async_copy(
                rows_v, out_hbm.at[pl.ds(base + c * CHUNK, CHUNK)], wsem)

        def iter_body(i, carry):
            ca = 2 * i
            cb = 2 * i + 1
            g_a = fire(ca, idx_a, rows_a, gsem_a)
            g_b = fire(cb, idx_b, rows_b, gsem_b)
            for cp in g_a:
                cp.wait()
            w_a = add_store(ca, rows_a, wsem_a)
            for cp in g_b:
                cp.wait()
            w_b = add_store(cb, rows_b, wsem_b)
            w_a.wait()
            w_b.wait()
            return carry

        lax.fori_loop(0, nchunk // 2, iter_body, 0)

    return emb


def kernel(seq, token_table, pos_table):
    B, L = seq.shape
    D = token_table.shape[1]
    seq2d = seq.astype(jnp.int32).reshape(B * L // GROUP, GROUP)
    out = _build(B, L, D)(seq2d, token_table, pos_table)
    return out.reshape(B, L, D)


# speedup vs baseline: 1.1715x; 1.1715x over previous
import functools

import jax
import jax.numpy as jnp
from jax import lax
from jax.experimental import pallas as pl
from jax.experimental.pallas import tpu as pltpu
from jax.experimental.pallas import tpu_sc as plsc

NC = 2
NS = 16
NW = NC * NS
LANES = 16
GROUP = 128
CHUNK = 512
GPC = CHUNK // GROUP


def _build(B, L, D):
    flat = B * L
    per_w = flat // NW
    nchunk = per_w // CHUNK
    rep = L * -(-(CHUNK + L) // L)
    half = D // LANES

    mesh = plsc.VectorSubcoreMesh(core_axis_name="c", subcore_axis_name="s")

    @functools.partial(
        pl.kernel,
        out_type=jax.ShapeDtypeStruct((flat, D), jnp.float32),
        mesh=mesh,
        compiler_params=pltpu.CompilerParams(use_tc_tiling_on_sc=False),
        scratch_types=[
            pltpu.VMEM((GPC, GROUP), jnp.int32),
            pltpu.VMEM((GPC, GROUP), jnp.int32),
            pltpu.VMEM((CHUNK, D), jnp.float32),
            pltpu.VMEM((CHUNK, D), jnp.float32),
            pltpu.VMEM((rep, D), jnp.float32),
            pltpu.SemaphoreType.DMA,
            pltpu.SemaphoreType.DMA,
            pltpu.SemaphoreType.DMA,
            pltpu.SemaphoreType.DMA,
        ],
    )
    def emb(seq_hbm, tok_hbm, pos_hbm, out_hbm,
            idx_a, idx_b, rows_a, rows_b, posrep,
            gsem_a, gsem_b, wsem_a, wsem_b):
        wid = lax.axis_index("s") * NC + lax.axis_index("c")
        row0 = wid * (per_w // GROUP)
        base = wid * per_w

        for j in range(rep // L):
            pltpu.sync_copy(pos_hbm, posrep.at[pl.ds(j * L, L)])

        def fire(c, idx_v, rows_v, gsem):
            pltpu.sync_copy(seq_hbm.at[pl.ds(row0 + c * GPC, GPC)], idx_v)
            return [
                pltpu.async_copy(tok_hbm.at[idx_v.at[g]],
                                 rows_v.at[pl.ds(g * GROUP, GROUP)], gsem)
                for g in range(GPC)
            ]

        def add_store(c, rows_v, wsem):
            p = lax.rem(c * CHUNK, L)

            def ab(i, carry):
                for h in range(half):
                    sl = pl.ds(h * LANES, LANES)
                    rows_v[i, sl] = rows_v[i, sl] + posrep[p + i, sl]
                return carry

            lax.fori_loop(0, CHUNK, ab, 0)
            return pltpu.async_copy(
                rows_v, out_hbm.at[pl.ds(base + c * CHUNK, CHUNK)], wsem)

        def iter_body(i, carry):
            ca = 2 * i
            cb = 2 * i + 1
            g_a = fire(ca, idx_a, rows_a, gsem_a)
            g_b = fire(cb, idx_b, rows_b, gsem_b)
            for cp in g_a:
                cp.wait()
            w_a = add_store(ca, rows_a, wsem_a)
            for cp in g_b:
                cp.wait()
            w_b = add_store(cb, rows_b, wsem_b)
            w_a.wait()
            w_b.wait()
            return carry

        lax.fori_loop(0, nchunk // 2, iter_body, 0)

    return emb


def kernel(seq, token_table, pos_table):
    B, L = seq.shape
    D = token_table.shape[1]
    seq2d = seq.astype(jnp.int32).reshape(B * L // GROUP, GROUP)
    out = _build(B, L, D)(seq2d, token_table, pos_table)
    return out.reshape(B, L, D)


# speedup vs baseline: 1.4696x; 1.2545x over previous
import functools

import jax
import jax.numpy as jnp
from jax import lax
from jax.experimental import pallas as pl
from jax.experimental.pallas import tpu as pltpu
from jax.experimental.pallas import tpu_sc as plsc

NC = 2
NS = 16
NW = NC * NS
LANES = 16
BBLK = 512
GROUP = 128


def _build(B, L, D):
    nblk = B // BBLK
    items = L * nblk
    per_w = items // NW
    gpc = BBLK // GROUP
    ngrp = BBLK // LANES

    mesh = plsc.VectorSubcoreMesh(core_axis_name="c", subcore_axis_name="s")

    @functools.partial(
        pl.kernel,
        out_type=jax.ShapeDtypeStruct((L, D, B), jnp.float32),
        mesh=mesh,
        compiler_params=pltpu.CompilerParams(
            use_tc_tiling_on_sc=False, needs_layout_passes=False),
        scratch_types=[
            pltpu.VMEM((BBLK,), jnp.int32),
            pltpu.VMEM((BBLK,), jnp.int32),
            pltpu.VMEM((BBLK, D), jnp.float32),
            pltpu.VMEM((BBLK, D), jnp.float32),
            pltpu.VMEM((D, BBLK), jnp.float32),
            pltpu.VMEM((D, BBLK), jnp.float32),
            pltpu.VMEM((L, D), jnp.float32),
            pltpu.SemaphoreType.DMA,
            pltpu.SemaphoreType.DMA,
            pltpu.SemaphoreType.DMA,
            pltpu.SemaphoreType.DMA,
        ],
    )
    def emb(seq_hbm, tok_hbm, pos_hbm, out_hbm,
            idx_a, idx_b, rows_a, rows_b, tout_a, tout_b, pos_v,
            gsem_a, gsem_b, wsem_a, wsem_b):
        wid = lax.axis_index("s") * NC + lax.axis_index("c")
        item0 = wid * per_w

        pltpu.sync_copy(pos_hbm, pos_v)

        def fire(t, idx_v, rows_v, gsem):
            l = t // nblk
            bb = t % nblk
            pltpu.sync_copy(seq_hbm.at[l, pl.ds(bb * BBLK, BBLK)], idx_v)
            return [
                pltpu.async_copy(tok_hbm.at[idx_v.at[pl.ds(g * GROUP, GROUP)]],
                                 rows_v.at[pl.ds(g * GROUP, GROUP)], gsem)
                for g in range(gpc)
            ]

        def transpose_add(t, rows_v, tout_v):
            l = t // nblk
            pvec = [pos_v[l, pl.ds(h * LANES, LANES)] for h in range(D // LANES)]
            pos_row = [pvec[d // LANES][d % LANES] for d in range(D)]

            @plsc.parallel_loop(0, ngrp)
            def _(g):
                ridx = g * LANES + lax.iota(jnp.int32, LANES)
                for d in range(D):
                    cidx = jnp.full((LANES,), d, jnp.int32)
                    vals = plsc.load_gather(rows_v, [ridx, cidx])
                    tout_v[d, pl.ds(g * LANES, LANES)] = vals + pos_row[d]

        def out_slice(t):
            l = t // nblk
            bb = t % nblk
            return out_hbm.at[l, :, pl.ds(bb * BBLK, BBLK)]

        def iter_body(i, carry):
            ta = item0 + 2 * i
            tb = item0 + 2 * i + 1
            g_a = fire(ta, idx_a, rows_a, gsem_a)
            g_b = fire(tb, idx_b, rows_b, gsem_b)
            for cp in g_a:
                cp.wait()
            transpose_add(ta, rows_a, tout_a)

            @pl.when(i > 0)
            def _():
                pltpu.make_async_copy(tout_a, out_slice(ta), wsem_a).wait()

            pltpu.async_copy(tout_a, out_slice(ta), wsem_a)
            for cp in g_b:
                cp.wait()
            transpose_add(tb, rows_b, tout_b)

            @pl.when(i > 0)
            def _():
                pltpu.make_async_copy(tout_b, out_slice(tb), wsem_b).wait()

            pltpu.async_copy(tout_b, out_slice(tb), wsem_b)
            return carry

        lax.fori_loop(0, per_w // 2, iter_body, 0)
        pltpu.make_async_copy(tout_a, out_slice(item0), wsem_a).wait()
        pltpu.make_async_copy(tout_b, out_slice(item0), wsem_b).wait()

    return emb


def kernel(seq, token_table, pos_table):
    B, L = seq.shape
    D = token_table.shape[1]
    seq_t = jnp.swapaxes(seq, 0, 1).astype(jnp.int32)
    out3 = _build(B, L, D)(seq_t, token_table, pos_table)
    return jnp.transpose(out3, (2, 0, 1))
